# baseline (speedup 1.0000x reference)
import functools

import numpy as np
import jax
import jax.numpy as jnp
from jax.experimental import pallas as pl
from jax.experimental.pallas import tpu as pltpu


def _bilinear_matrix_np(n_in, n_out):
    src = np.arange(n_out, dtype=np.float64) * (n_in - 1) / (n_out - 1)
    i0 = np.clip(np.floor(src).astype(np.int64), 0, n_in - 1)
    i1 = np.clip(i0 + 1, 0, n_in - 1)
    w1 = src - i0
    w0 = 1.0 - w1
    A = np.zeros((n_out, n_in), np.float64)
    rows = np.arange(n_out)
    A[rows, i0] += w0
    A[rows, i1] += w1
    return A


def _fused_kernel(x_ref, res_ref, mt_ref, w1_ref, s1_ref, b1_ref,
                  w2_ref, s2_ref, b2_ref, o_ref, *, W1, S1, B, Cx):
    def shifted(v, off):
        return v if off == 0 else pltpu.roll(v, S1 - off, 1)

    def im2col(v):
        return jnp.concatenate(
            [shifted(v, kh * W1 + kw) for kh in range(3) for kw in range(3)],
            axis=0)

    up = jnp.dot(x_ref[0], mt_ref[...],
                 preferred_element_type=jnp.float32).astype(jnp.bfloat16)

    for i in range(B):
        v = jnp.concatenate([res_ref[i], up[i * Cx:(i + 1) * Cx]], axis=0)
        acc1 = jnp.dot(w1_ref[...], im2col(v),
                       preferred_element_type=jnp.float32)
        y1 = jnp.maximum(acc1 * s1_ref[...] + b1_ref[...],
                         0.0).astype(jnp.bfloat16)

        acc2 = jnp.dot(w2_ref[...], im2col(y1),
                       preferred_element_type=jnp.float32)
        y2 = jnp.maximum(acc2 * s2_ref[...] + b2_ref[...], 0.0)
        o_ref[i] = y2[:, :o_ref.shape[2]].astype(o_ref.dtype)


def kernel(x, residual, w1, scale1, bias1, w2, scale2, bias2):
    N, Cx, H, W = x.shape
    Cr, Hr, Wr = residual.shape[1], residual.shape[2], residual.shape[3]
    H1, W1 = 2 * H, 2 * W
    S1 = H1 * W1
    dy, dx = (Hr - H1) // 2, (Wr - W1) // 2
    C1, C2 = w1.shape[1], w2.shape[1]
    B = 8 if N % 8 == 0 else (4 if N % 4 == 0 else 1)

    ah = _bilinear_matrix_np(H, H1)
    aw = _bilinear_matrix_np(W, W1)
    mt = jnp.asarray(np.kron(ah, aw).T, jnp.bfloat16)

    xf = x.reshape(N // B, B * Cx, H * W).astype(jnp.bfloat16)
    res = residual.astype(jnp.bfloat16)[:, :, dy:Hr - dy, dx:Wr - dx]
    res = res.reshape(N, Cr, S1)
    w1m = w1.transpose(1, 0, 2).reshape(C1, 9 * (Cr + Cx)).astype(jnp.bfloat16)
    w2m = w2.transpose(1, 0, 2).reshape(C2, 9 * C1).astype(jnp.bfloat16)
    s1 = scale1.reshape(C1, 1)
    b1 = bias1.reshape(C1, 1)
    s2 = scale2.reshape(C2, 1)
    b2 = bias2.reshape(C2, 1)

    So = (H1 - 4) * W1
    fn = functools.partial(_fused_kernel, W1=W1, S1=S1, B=B, Cx=Cx)
    out = pl.pallas_call(
        fn,
        out_shape=jax.ShapeDtypeStruct((N, C2, So), jnp.bfloat16),
        grid=(N // B,),
        in_specs=[
            pl.BlockSpec((1, B * Cx, H * W), lambda n: (n, 0, 0)),
            pl.BlockSpec((B, Cr, S1), lambda n: (n, 0, 0)),
            pl.BlockSpec((H * W, S1), lambda n: (0, 0)),
            pl.BlockSpec((C1, 9 * (Cr + Cx)), lambda n: (0, 0)),
            pl.BlockSpec((C1, 1), lambda n: (0, 0)),
            pl.BlockSpec((C1, 1), lambda n: (0, 0)),
            pl.BlockSpec((C2, 9 * C1), lambda n: (0, 0)),
            pl.BlockSpec((C2, 1), lambda n: (0, 0)),
            pl.BlockSpec((C2, 1), lambda n: (0, 0)),
        ],
        out_specs=pl.BlockSpec((B, C2, So), lambda n: (n, 0, 0)),
        compiler_params=pltpu.CompilerParams(
            dimension_semantics=("arbitrary",)),
    )(xf, res, mt, w1m, s1, b1, w2m, s2, b2)
    out = out.reshape(N, C2, H1 - 4, W1)[:, :, :, :W1 - 4]
    return out.astype(x.dtype)


# speedup vs baseline: 1.0879x; 1.0879x over previous
import functools

import numpy as np
import jax
import jax.numpy as jnp
from jax.experimental import pallas as pl
from jax.experimental.pallas import tpu as pltpu


def _bilinear_matrix_np(n_in, n_out):
    src = np.arange(n_out, dtype=np.float64) * (n_in - 1) / (n_out - 1)
    i0 = np.clip(np.floor(src).astype(np.int64), 0, n_in - 1)
    i1 = np.clip(i0 + 1, 0, n_in - 1)
    w1 = src - i0
    w0 = 1.0 - w1
    A = np.zeros((n_out, n_in), np.float64)
    rows = np.arange(n_out)
    A[rows, i0] += w0
    A[rows, i1] += w1
    return A


def _fused_kernel(x_ref, res_ref, mt_ref, w1_ref, s1_ref, b1_ref,
                  w2_ref, s2_ref, b2_ref, o_ref, *, W1, S1, B, Cx):
    def shifted(v, off):
        return v if off == 0 else pltpu.roll(v, S1 - off, 1)

    def im2col(v):
        return jnp.concatenate(
            [shifted(v, kh * W1 + kw) for kh in range(3) for kw in range(3)],
            axis=0)

    up = jnp.dot(x_ref[0], mt_ref[...],
                 preferred_element_type=jnp.float32).astype(jnp.bfloat16)

    for i in range(B):
        v = jnp.concatenate([res_ref[i], up[i * Cx:(i + 1) * Cx]], axis=0)
        acc1 = jnp.dot(w1_ref[...], im2col(v),
                       preferred_element_type=jnp.float32)
        y1 = jnp.maximum(acc1 * s1_ref[...] + b1_ref[...],
                         0.0).astype(jnp.bfloat16)

        acc2 = jnp.dot(w2_ref[...], im2col(y1),
                       preferred_element_type=jnp.float32)
        y2 = jnp.maximum(acc2 * s2_ref[...] + b2_ref[...], 0.0)
        o_ref[i] = y2[:, :o_ref.shape[2]].astype(o_ref.dtype)


def kernel(x, residual, w1, scale1, bias1, w2, scale2, bias2):
    N, Cx, H, W = x.shape
    Cr, Hr, Wr = residual.shape[1], residual.shape[2], residual.shape[3]
    H1, W1 = 2 * H, 2 * W
    S1 = H1 * W1
    dy, dx = (Hr - H1) // 2, (Wr - W1) // 2
    C1, C2 = w1.shape[1], w2.shape[1]
    B = 4 if N % 4 == 0 else 1

    ah = _bilinear_matrix_np(H, H1)
    aw = _bilinear_matrix_np(W, W1)
    mt = jnp.asarray(np.kron(ah, aw).T, jnp.bfloat16)

    xf = x.reshape(N // B, B * Cx, H * W).astype(jnp.bfloat16)
    res = residual.astype(jnp.bfloat16)[:, :, dy:Hr - dy, dx:Wr - dx]
    res = res.reshape(N, Cr, S1)
    w1m = w1.transpose(1, 0, 2).reshape(C1, 9 * (Cr + Cx)).astype(jnp.bfloat16)
    w2m = w2.transpose(1, 0, 2).reshape(C2, 9 * C1).astype(jnp.bfloat16)
    s1 = scale1.reshape(C1, 1)
    b1 = bias1.reshape(C1, 1)
    s2 = scale2.reshape(C2, 1)
    b2 = bias2.reshape(C2, 1)

    So = (H1 - 4) * W1
    fn = functools.partial(_fused_kernel, W1=W1, S1=S1, B=B, Cx=Cx)
    out = pl.pallas_call(
        fn,
        out_shape=jax.ShapeDtypeStruct((N, C2, So), jnp.bfloat16),
        grid=(N // B,),
        in_specs=[
            pl.BlockSpec((1, B * Cx, H * W), lambda n: (n, 0, 0)),
            pl.BlockSpec((B, Cr, S1), lambda n: (n, 0, 0)),
            pl.BlockSpec((H * W, S1), lambda n: (0, 0)),
            pl.BlockSpec((C1, 9 * (Cr + Cx)), lambda n: (0, 0)),
            pl.BlockSpec((C1, 1), lambda n: (0, 0)),
            pl.BlockSpec((C1, 1), lambda n: (0, 0)),
            pl.BlockSpec((C2, 9 * C1), lambda n: (0, 0)),
            pl.BlockSpec((C2, 1), lambda n: (0, 0)),
            pl.BlockSpec((C2, 1), lambda n: (0, 0)),
        ],
        out_specs=pl.BlockSpec((B, C2, So), lambda n: (n, 0, 0)),
        compiler_params=pltpu.CompilerParams(
            dimension_semantics=("arbitrary",)),
    )(xf, res, mt, w1m, s1, b1, w2m, s2, b2)
    out = out.reshape(N, C2, H1 - 4, W1)[:, :, :, :W1 - 4]
    return out.astype(x.dtype)
